# baseline (speedup 1.0000x reference)
import jax
import jax.numpy as jnp
from jax.experimental import pallas as pl
import jax.experimental.pallas.tpu as pltpu

E = 64
TOP_K = 2
D_MODEL = 128
BATCH = 64
SEQ_LEN = 4096

SEQ_BLK = 512
N_BLKS = SEQ_LEN // SEQ_BLK


def _router_kernel(x_ref, c_ref, scores_out_ref, idx_out_ref, acc_ref):
    step = pl.program_id(0)

    @pl.when(step == 0)
    def _init():
        acc_ref[...] = jnp.zeros_like(acc_ref)

    acc_ref[...] += jnp.sum(x_ref[...], axis=1)

    @pl.when(step == N_BLKS - 1)
    def _finalize():
        q = acc_ref[...] * (1.0 / SEQ_LEN)
        qn = jnp.sqrt(jnp.sum(q * q, axis=1, keepdims=True))
        q = q / jnp.maximum(qn, 1e-12)

        c = c_ref[...]
        cn = jnp.sqrt(jnp.sum(c * c, axis=1, keepdims=True))
        c = c / jnp.maximum(cn, 1e-12)

        scores = jax.lax.dot_general(
            q, c, (((1,), (1,)), ((), ())), preferred_element_type=jnp.float32
        )

        iota = jax.lax.broadcasted_iota(jnp.int32, (BATCH, E), 1)
        m1 = jnp.max(scores, axis=1, keepdims=True)
        i1 = jnp.min(
            jnp.where(scores == m1, iota, jnp.int32(2**30)), axis=1, keepdims=True
        )
        masked = jnp.where(iota == i1, -jnp.inf, scores)
        m2 = jnp.max(masked, axis=1, keepdims=True)
        i2 = jnp.min(
            jnp.where(masked == m2, iota, jnp.int32(2**30)), axis=1, keepdims=True
        )

        scores_out_ref[:, 0:1] = m1
        scores_out_ref[:, 1:2] = m2
        idx_out_ref[:, 0:1] = i1
        idx_out_ref[:, 1:2] = i2


@jax.jit
def kernel(x, centroids):
    top_scores, top_idx = pl.pallas_call(
        _router_kernel,
        grid=(N_BLKS,),
        in_specs=[
            pl.BlockSpec((BATCH, SEQ_BLK, D_MODEL), lambda i: (0, i, 0)),
            pl.BlockSpec((E, D_MODEL), lambda i: (0, 0)),
        ],
        out_specs=[
            pl.BlockSpec((BATCH, TOP_K), lambda i: (0, 0)),
            pl.BlockSpec((BATCH, TOP_K), lambda i: (0, 0)),
        ],
        out_shape=[
            jax.ShapeDtypeStruct((BATCH, TOP_K), jnp.float32),
            jax.ShapeDtypeStruct((BATCH, TOP_K), jnp.int32),
        ],
        scratch_shapes=[pltpu.VMEM((BATCH, D_MODEL), jnp.float32)],
    )(x, centroids)
    return top_scores, top_idx


# speedup vs baseline: 1.0048x; 1.0048x over previous
import jax
import jax.numpy as jnp
from jax.experimental import pallas as pl
import jax.experimental.pallas.tpu as pltpu

E = 64
TOP_K = 2
D_MODEL = 128
BATCH = 64
SEQ_LEN = 4096

B_BLK = 8
N_BLKS = BATCH // B_BLK


def _router_kernel(x_ref, c_ref, scores_out_ref, idx_out_ref):
    q = jnp.sum(x_ref[...], axis=1) * (1.0 / SEQ_LEN)
    qn = jnp.sqrt(jnp.sum(q * q, axis=1, keepdims=True))
    q = q / jnp.maximum(qn, 1e-12)

    c = c_ref[...]
    cn = jnp.sqrt(jnp.sum(c * c, axis=1, keepdims=True))
    c = c / jnp.maximum(cn, 1e-12)

    scores = jax.lax.dot_general(
        q, c, (((1,), (1,)), ((), ())), preferred_element_type=jnp.float32
    )

    iota = jax.lax.broadcasted_iota(jnp.int32, (B_BLK, E), 1)
    m1 = jnp.max(scores, axis=1, keepdims=True)
    i1 = jnp.min(
        jnp.where(scores == m1, iota, jnp.int32(2**30)), axis=1, keepdims=True
    )
    masked = jnp.where(iota == i1, -jnp.inf, scores)
    m2 = jnp.max(masked, axis=1, keepdims=True)
    i2 = jnp.min(
        jnp.where(masked == m2, iota, jnp.int32(2**30)), axis=1, keepdims=True
    )

    scores_out_ref[:, 0:1] = m1
    scores_out_ref[:, 1:2] = m2
    idx_out_ref[:, 0:1] = i1
    idx_out_ref[:, 1:2] = i2


@jax.jit
def kernel(x, centroids):
    top_scores, top_idx = pl.pallas_call(
        _router_kernel,
        grid=(N_BLKS,),
        in_specs=[
            pl.BlockSpec((B_BLK, SEQ_LEN, D_MODEL), lambda i: (i, 0, 0)),
            pl.BlockSpec((E, D_MODEL), lambda i: (0, 0)),
        ],
        out_specs=[
            pl.BlockSpec((B_BLK, TOP_K), lambda i: (i, 0)),
            pl.BlockSpec((B_BLK, TOP_K), lambda i: (i, 0)),
        ],
        out_shape=[
            jax.ShapeDtypeStruct((BATCH, TOP_K), jnp.float32),
            jax.ShapeDtypeStruct((BATCH, TOP_K), jnp.int32),
        ],
        compiler_params=pltpu.CompilerParams(
            dimension_semantics=("arbitrary",),
        ),
    )(x, centroids)
    return top_scores, top_idx
